# baseline (speedup 1.0000x reference)
import functools

import jax
import jax.numpy as jnp
from jax import lax
from jax.experimental import pallas as pl
from jax.experimental.pallas import tpu as pltpu
from jax.experimental.pallas import tpu_sc as plsc

_NUM_SEGMENTS = 10000
_SEG_HALF = 5120
_ACC_ROWS = _SEG_HALF + 8
_N_TOKENS = 320000
_D = 128
_SUB = 128
_BLOCK = 256
_NSUB = _BLOCK // _SUB
_NBLOCKS = _N_TOKENS // _BLOCK
_NC = 2
_NS = 16
_RPT = _SEG_HALF // _NS


_mesh = plsc.VectorSubcoreMesh(core_axis_name="c", subcore_axis_name="s")


@functools.partial(
    pl.kernel,
    mesh=_mesh,
    out_type=jax.ShapeDtypeStruct((_NUM_SEGMENTS, _D), jnp.float32),
    scratch_types=[
        pltpu.VMEM((2, _NSUB, _SUB), jnp.int32),
        pltpu.VMEM((2, _BLOCK, _D), jnp.float32),
        pltpu.VMEM((_SUB,), jnp.float32),
        pltpu.VMEM((32, _D), jnp.float32),
        pltpu.VMEM((_RPT,), jnp.float32),
        pltpu.VMEM((16,), jnp.int32),
        pltpu.VMEM_SHARED((_ACC_ROWS, _D), jnp.float32),
        pltpu.VMEM_SHARED((_ACC_ROWS,), jnp.float32),
        pltpu.SemaphoreType.DMA((2,)),
        pltpu.SemaphoreType.DMA((2,)),
    ],
)
def _sc_mean(enc_hbm, ids_hbm, tlo_hbm, out_hbm,
             idx_v, rows_v, ones_v, zero_v, cnt_v, tlo_v, acc_sh, cnt_sh,
             sem_rows, sem_ids):
    cid = lax.axis_index("c")
    sid = lax.axis_index("s")

    pltpu.sync_copy(tlo_hbm, tlo_v)

    for j in range(_SUB // 16):
        ones_v[pl.ds(j * 16, 16)] = jnp.ones((16,), jnp.float32)

    def zrow(r, carry):
        for j in range(_D // 16):
            zero_v[r, pl.ds(j * 16, 16)] = jnp.zeros((16,), jnp.float32)
        return carry

    lax.fori_loop(0, 32, zrow, 0)

    base_row = sid * _RPT

    def zacc(t, carry):
        pltpu.sync_copy(zero_v, acc_sh.at[pl.ds(base_row + t * 32, 32)])
        return carry

    lax.fori_loop(0, _RPT // 32, zacc, 0)
    pltpu.sync_copy(zero_v.at[0], cnt_sh.at[pl.ds(base_row, 128)])
    pltpu.sync_copy(zero_v.at[0], cnt_sh.at[pl.ds(base_row + 128, 128)])
    pltpu.sync_copy(zero_v.at[0, pl.ds(0, 64)],
                    cnt_sh.at[pl.ds(base_row + 256, 64)])

    plsc.subcore_barrier()

    t_lo = tlo_v[...][0]
    lo = jnp.where(cid == 0, 0, t_lo // _BLOCK)
    hi = jnp.where(cid == 0, (t_lo + _BLOCK - 1) // _BLOCK, _NBLOCKS)
    n_c = hi - lo
    per = n_c // _NS
    rem = n_c - per * _NS
    base = lo + sid * per + jnp.minimum(sid, rem)
    n_my = per + jnp.where(sid < rem, 1, 0)
    seg_base = cid * _SEG_HALF

    def _start_load(c, b):
        pltpu.async_copy(enc_hbm.at[pl.ds(c * _BLOCK, _BLOCK)], rows_v.at[b],
                         sem_rows.at[b])
        pltpu.async_copy(ids_hbm.at[c], idx_v.at[b], sem_ids.at[b])

    def _wait_load(c, b):
        pltpu.make_async_copy(enc_hbm.at[pl.ds(c * _BLOCK, _BLOCK)],
                              rows_v.at[b], sem_rows.at[b]).wait()
        pltpu.make_async_copy(ids_hbm.at[c], idx_v.at[b],
                              sem_ids.at[b]).wait()

    @pl.when(n_my > 0)
    def _prime():
        _start_load(base, 0)

    def body(i, carry):
        b = i % 2

        @pl.when(i + 1 < n_my)
        def _next():
            _start_load(base + i + 1, (i + 1) % 2)

        _wait_load(base + i, b)
        for j in range(_NSUB):
            for k in range(_SUB // 16):
                v = idx_v[b, j, pl.ds(k * 16, 16)] - seg_base
                oob = (v < 0) | (v >= _SEG_HALF)
                idx_v[b, j, pl.ds(k * 16, 16)] = jnp.where(oob, _SEG_HALF, v)
            idx_row = idx_v.at[b, j]
            pltpu.sync_copy(rows_v.at[b, pl.ds(j * _SUB, _SUB)],
                            acc_sh.at[idx_row], add=True)
            pltpu.sync_copy(ones_v, cnt_sh.at[idx_row], add=True)
        return carry

    lax.fori_loop(0, n_my, body, 0)

    plsc.subcore_barrier()

    pltpu.sync_copy(cnt_sh.at[pl.ds(base_row, _RPT)], cnt_v)

    def recip(k, carry):
        cv = cnt_v[pl.ds(k * 16, 16)]
        cnt_v[pl.ds(k * 16, 16)] = 1.0 / jnp.maximum(cv, 1.0)
        return carry

    lax.fori_loop(0, _RPT // 16, recip, 0)

    seg0 = seg_base + base_row
    for p in range(2):
        pltpu.sync_copy(acc_sh.at[pl.ds(base_row + p * 160, 160)],
                        rows_v.at[0, pl.ds(0, 160)])

        def divgrp(g, carry):
            m16 = cnt_v[pl.ds(p * 160 + g * 16, 16)]
            for rr in range(16):
                r = g * 16 + rr
                m = lax.broadcast(m16[rr], (16,))
                for k in range(_D // 16):
                    rows_v[0, r, pl.ds(k * 16, 16)] = (
                        rows_v[0, r, pl.ds(k * 16, 16)] * m)
            return carry

        lax.fori_loop(0, 10, divgrp, 0)

        n16 = jnp.clip(_NUM_SEGMENTS - (seg0 + p * 160), 0, 160) // 16

        def wout(t, carry):
            pltpu.sync_copy(rows_v.at[0, pl.ds(t * 16, 16)],
                            out_hbm.at[pl.ds(seg0 + p * 160 + t * 16, 16)])
            return carry

        lax.fori_loop(0, n16, wout, 0)


@jax.jit
def _impl(enc_seq, segment_ids):
    ids3d = segment_ids.reshape(_NBLOCKS, _NSUB, _SUB)
    t_lo = jnp.searchsorted(segment_ids, _SEG_HALF).astype(jnp.int32)
    tlo16 = jnp.broadcast_to(t_lo, (16,))
    return _sc_mean(enc_seq, ids3d, tlo16)


def kernel(enc_seq, segment_ids):
    return _impl(enc_seq, segment_ids)
